# baseline (speedup 1.0000x reference)
import functools

import jax
import jax.numpy as jnp
from jax import lax
from jax.experimental import pallas as pl
from jax.experimental.pallas import tpu as pltpu
from jax.experimental.pallas import tpu_sc as plsc

B = 4096
DEG = 16
NSAMP = 10
S1 = NSAMP + 1
D = 128
N_NODES = 100000

NC = 2
NSUB = 16
NWORK = NC * NSUB
BPW = B // NWORK
IDS_PW = BPW * S1
NT_PACK = N_NODES * DEG // D

HALF = N_NODES // NC
TILE_IDS = 3136
HALF_PAD = NSUB * TILE_IDS
DUMP_BASE = HALF

ROWS_PC = 8
IDS_PC = ROWS_PC * S1
NCHUNK = BPW // ROWS_PC

WS_PT = (B * S1) // NSUB
TOTAL = B * S1

_params = pltpu.CompilerParams(needs_layout_passes=False)
_mesh = lambda: plsc.VectorSubcoreMesh(
    core_axis_name="c", subcore_axis_name="s", num_cores=NC, num_subcores=NSUB)


def _wid():
    return lax.axis_index("s") * NC + lax.axis_index("c")


def _k1_body(bn_hbm, nt2_hbm, ruf_hbm, x_hbm,
             ws_hbm, agg_hbm,
             bn_v, idxb_v, packed_v, rand_v, ws_v, xrows_v, agg_v,
             sem0, sem1):
    wid = _wid()
    base = wid * BPW
    iota16 = lax.iota(jnp.int32, 16)

    pltpu.sync_copy(bn_hbm.at[pl.ds(base, BPW)], bn_v)
    pltpu.sync_copy(ruf_hbm.at[pl.ds(base * DEG, BPW * DEG)], rand_v)
    for g in range(BPW // 16):
        bn_g = bn_v[pl.ds(g * 16, 16)]
        idxb_v[pl.ds(g * 16, 16)] = lax.shift_right_logical(bn_g, 3)
    pltpu.async_copy(nt2_hbm.at[idxb_v], packed_v, sem0).wait()

    inv = jnp.float32(1.0 / S1)
    sems = (sem0, sem1)

    def rank_rows(c):
        def rb(rr, carry):
            r = c * ROWS_PC + rr
            rvec = jnp.full((16,), r, jnp.int32)
            bnr = plsc.load_gather(bn_v, [rvec])
            lane = (bnr & 7) * DEG + iota16
            nb = plsc.load_gather(packed_v, [rvec, lane])
            u = plsc.load_gather(rand_v, [r * DEG + iota16])
            rank = jnp.zeros((16,), jnp.int32)
            for j in range(DEG):
                uj = jnp.broadcast_to(u[j], (16,))
                cond = (uj < u) | ((uj == u) & (iota16 > j))
                rank = rank + jnp.where(cond, 1, 0)
            pos = r * S1 + jnp.minimum(rank, S1 - 1)
            plsc.store_scatter(ws_v, [pos], nb, mask=rank < NSAMP)
            return carry
        lax.fori_loop(0, ROWS_PC, rb, 0)
        rows = c * ROWS_PC + iota16
        vals = plsc.load_gather(bn_v, [jnp.minimum(rows, BPW - 1)])
        plsc.store_scatter(ws_v, [rows * S1 + NSAMP], vals,
                           mask=iota16 < ROWS_PC)

    def fire(c, b):
        return pltpu.async_copy(
            x_hbm.at[ws_v.at[pl.ds(c * IDS_PC, IDS_PC)]], xrows_v.at[b],
            sems[b])

    def accum(c, b):
        def ab(rr, carry):
            for v in range(D // 16):
                acc = xrows_v[b, rr * S1, pl.ds(v * 16, 16)]
                for k in range(1, S1):
                    acc = acc + xrows_v[b, rr * S1 + k, pl.ds(v * 16, 16)]
                agg_v[c * ROWS_PC + rr, pl.ds(v * 16, 16)] = acc * inv
            return carry
        lax.fori_loop(0, ROWS_PC, ab, 0)

    rank_rows(0)
    fire(0, 0)

    def chunk_body(g, carry):
        for b in range(2):
            c = 2 * g + b
            nxt = c + 1

            @pl.when(nxt < NCHUNK)
            def _():
                rank_rows(nxt)
                fire(nxt, (b + 1) % 2)
            pltpu.make_async_copy(
                x_hbm.at[ws_v.at[pl.ds(c * IDS_PC, IDS_PC)]], xrows_v.at[b],
                sems[b]).wait()
            accum(c, b)
        return carry

    lax.fori_loop(0, NCHUNK // 2, chunk_body, 0)
    pltpu.sync_copy(ws_v, ws_hbm.at[pl.ds(base * S1, IDS_PW)])
    pltpu.sync_copy(agg_v, agg_hbm.at[pl.ds(base, BPW)])


def _run_k1(batch_node, x, nt_packed, ru_flat):
    kfn = pl.kernel(
        _k1_body,
        out_type=(
            jax.ShapeDtypeStruct((TOTAL,), jnp.int32),
            jax.ShapeDtypeStruct((B, D), jnp.float32),
        ),
        mesh=_mesh(),
        compiler_params=_params,
        scratch_types=[
            pltpu.VMEM((BPW,), jnp.int32),
            pltpu.VMEM((BPW,), jnp.int32),
            pltpu.VMEM((BPW, D), jnp.int32),
            pltpu.VMEM((BPW * DEG,), jnp.float32),
            pltpu.VMEM((IDS_PW,), jnp.int32),
            pltpu.VMEM((2, IDS_PC, D), jnp.float32),
            pltpu.VMEM((BPW, D), jnp.float32),
            pltpu.SemaphoreType.DMA,
            pltpu.SemaphoreType.DMA,
        ],
    )
    return kfn(batch_node, nt_packed, ru_flat, x)


N_SCHUNK = WS_PT // 128


def _k2_body(ws_hbm, cmp_hbm, cnt_hbm,
             flags_sp, ws_v, idx2_v, ones_v, fbuf, cbuf, part_v, sem0):
    cid = lax.axis_index("c")
    sid = lax.axis_index("s")
    iota16 = lax.iota(jnp.int32, 16)
    lo = cid * HALF

    def zfill(g, carry):
        fbuf[pl.ds(g * 16, 16)] = jnp.zeros((16,), jnp.int32)
        return carry
    lax.fori_loop(0, TILE_IDS // 16, zfill, 0)
    pltpu.sync_copy(fbuf, flags_sp.at[pl.ds(sid * TILE_IDS, TILE_IDS)])

    pltpu.sync_copy(ws_hbm.at[pl.ds(sid * WS_PT, WS_PT)], ws_v)
    for g in range(WS_PT // 16):
        v = ws_v[pl.ds(g * 16, 16)]
        local = v - lo
        in_half = (local >= 0) & (local < HALF)
        dump = DUMP_BASE + (v & 127)
        idx2_v[g // 8, pl.ds((g % 8) * 16, 16)] = jnp.where(in_half, local,
                                                           dump)
    for g in range(8):
        ones_v[pl.ds(g * 16, 16)] = jnp.ones((16,), jnp.int32)

    plsc.subcore_barrier()
    for j in range(N_SCHUNK):
        pltpu.sync_copy(ones_v, flags_sp.at[idx2_v.at[j]], add=True)
    plsc.subcore_barrier()

    pltpu.sync_copy(flags_sp.at[pl.ds(sid * TILE_IDS, TILE_IDS)], fbuf)

    def cmp_body(g, run):
        f = fbuf[pl.ds(g * 16, 16)]
        gid = sid * TILE_IDS + g * 16 + iota16
        ind = jnp.where((f > 0) & (gid < HALF), 1, 0)
        incl = plsc.cumsum(ind)
        plsc.store_scatter(cbuf, [run + (incl - ind)], lo + gid,
                           mask=ind > 0)
        return run + jnp.sum(ind)
    my_cnt = lax.fori_loop(0, TILE_IDS // 16, cmp_body, jnp.int32(0))

    pltpu.sync_copy(cbuf, cmp_hbm.at[pl.ds((cid * NSUB + sid) * TILE_IDS,
                                           TILE_IDS)])
    part_v[...] = jnp.broadcast_to(my_cnt, (16,))
    pltpu.sync_copy(part_v, cnt_hbm.at[cid, pl.ds(sid * 16, 16)])


def _run_k2(ws_flat):
    kfn = pl.kernel(
        _k2_body,
        out_type=(
            jax.ShapeDtypeStruct((NC * HALF_PAD,), jnp.int32),
            jax.ShapeDtypeStruct((NC, NSUB * 16), jnp.int32),
        ),
        mesh=_mesh(),
        compiler_params=_params,
        scratch_types=[
            pltpu.VMEM_SHARED((HALF_PAD,), jnp.int32),
            pltpu.VMEM((WS_PT,), jnp.int32),
            pltpu.VMEM((N_SCHUNK, 128), jnp.int32),
            pltpu.VMEM((128,), jnp.int32),
            pltpu.VMEM((TILE_IDS,), jnp.int32),
            pltpu.VMEM((TILE_IDS,), jnp.int32),
            pltpu.VMEM((16,), jnp.int32),
            pltpu.SemaphoreType.DMA,
        ],
    )
    return kfn(ws_flat)


def _k3_body(cmp_hbm, cnt_hbm,
             out_hbm,
             cnt_v, cb_v, gidx_v, vals_v, out_v, sem0):
    wid = _wid()
    base = wid * IDS_PW
    iota16 = lax.iota(jnp.int32, 16)

    pltpu.sync_copy(cnt_hbm, cnt_v)

    cnt0 = jnp.zeros((16,), jnp.int32)
    cnt1 = jnp.zeros((16,), jnp.int32)
    for s in range(NSUB):
        r0 = cnt_v[0, pl.ds(s * 16, 16)]
        r1 = cnt_v[1, pl.ds(s * 16, 16)]
        cnt0 = cnt0 + jnp.where(iota16 == s, r0, 0)
        cnt1 = cnt1 + jnp.where(iota16 == s, r1, 0)

    incl0 = plsc.cumsum(cnt0)
    t0 = jnp.sum(cnt0)
    incl1 = plsc.cumsum(cnt1) + t0
    u_total = jnp.sum(cnt1) + t0

    plsc.store_scatter(cb_v, [iota16], jnp.zeros((16,), jnp.int32),
                       mask=iota16 < 1)
    plsc.store_scatter(cb_v, [iota16 + 1], incl0)
    plsc.store_scatter(cb_v, [iota16 + 17], incl1)

    for g in range(IDS_PW // 16):
        p = base + g * 16 + iota16
        t = jnp.zeros((16,), jnp.int32)
        for b in (16, 8, 4, 2, 1):
            cand = t + b
            cbv = plsc.load_gather(cb_v, [cand])
            t = jnp.where(cbv <= p, cand, t)
        cbt = plsc.load_gather(cb_v, [t])
        idx = t * TILE_IDS + (p - cbt)
        idx = jnp.where(p < u_total, idx, 0)
        gidx_v[g // 8, pl.ds((g % 8) * 16, 16)] = idx

    handles = [
        pltpu.async_copy(cmp_hbm.at[gidx_v.at[j]],
                         vals_v.at[pl.ds(j * 128, 128)], sem0)
        for j in range(IDS_PW // 128)
    ]
    for h in handles:
        h.wait()

    for g in range(IDS_PW // 16):
        p = base + g * 16 + iota16
        v = vals_v[pl.ds(g * 16, 16)]
        out_v[pl.ds(g * 16, 16)] = jnp.where(p < u_total, v, -1)

    pltpu.sync_copy(out_v, out_hbm.at[pl.ds(base, IDS_PW)])


def _run_k3(cmp_tab, cnt_tab):
    kfn = pl.kernel(
        _k3_body,
        out_type=jax.ShapeDtypeStruct((TOTAL,), jnp.int32),
        mesh=_mesh(),
        compiler_params=_params,
        scratch_types=[
            pltpu.VMEM((NC, NSUB * 16), jnp.int32),
            pltpu.VMEM((40,), jnp.int32),
            pltpu.VMEM((S1, 128), jnp.int32),
            pltpu.VMEM((IDS_PW,), jnp.int32),
            pltpu.VMEM((IDS_PW,), jnp.int32),
            pltpu.SemaphoreType.DMA,
        ],
    )
    return kfn(cmp_tab, cnt_tab)


@jax.jit
def kernel(batch_node, x, neighbor_table, rand_u):
    nt_packed = neighbor_table.reshape(NT_PACK, D)
    ru_flat = rand_u.reshape(-1)
    ws_flat, agg = _run_k1(batch_node, x, nt_packed, ru_flat)
    cmp_tab, cnt_tab = _run_k2(ws_flat)
    all_node = _run_k3(cmp_tab, cnt_tab)
    return ws_flat.reshape(B, S1), all_node, agg
